# Initial kernel scaffold; baseline (speedup 1.0000x reference)
import jax
import jax.numpy as jnp
from jax.experimental import pallas as pl


def kernel(src_ids, dst_ids, W1, b1, W2, b2):
    raise NotImplementedError("write your pallas kernel here")


# speedup vs baseline: 1.2312x; 1.2312x over previous
import jax
import jax.numpy as jnp
from jax.experimental import pallas as pl
from jax.experimental.pallas import tpu as pltpu

B, L, D = 256, 512, 128
BB = 8


def _kernel(s_ref, d_ref, w1_ref, b1_ref, w2_ref, b2_ref, out_s_ref, out_d_ref):
    w1 = w1_ref[...]
    b1 = b1_ref[...]
    w2 = w2_ref[...]
    b2x2 = 2.0 * b2_ref[...]

    def counts(col, row):
        m = col == row
        return jnp.sum(jnp.where(m, 1.0, 0.0), axis=1, keepdims=True)

    for r in range(BB):
        srow = s_ref[r:r + 1, :]
        drow = d_ref[r:r + 1, :]
        scol = jnp.transpose(srow)
        dcol = jnp.transpose(drow)

        c_ss = counts(scol, srow)
        c_sd = counts(scol, drow)
        c_ds = counts(dcol, srow)
        c_dd = counts(dcol, drow)

        smask = scol == 0
        dmask = dcol == 0
        c_ss = jnp.where(smask, 0.0, c_ss)
        c_sd = jnp.where(smask, 0.0, c_sd)
        c_ds = jnp.where(dmask, 0.0, c_ds)
        c_dd = jnp.where(dmask, 0.0, c_dd)

        u_s = jax.nn.relu(c_ss * w1 + b1) + jax.nn.relu(c_sd * w1 + b1)
        u_d = jax.nn.relu(c_ds * w1 + b1) + jax.nn.relu(c_dd * w1 + b1)

        out_s_ref[r] = jnp.dot(u_s, w2, preferred_element_type=jnp.float32) + b2x2
        out_d_ref[r] = jnp.dot(u_d, w2, preferred_element_type=jnp.float32) + b2x2


def kernel(src_ids, dst_ids, W1, b1, W2, b2):
    b1r = b1.reshape(1, D)
    b2r = b2.reshape(1, D)
    grid = (B // BB,)
    out_shape = jax.ShapeDtypeStruct((B, L, D), jnp.float32)
    wspec = pl.BlockSpec((1, D), lambda i: (0, 0))
    src_feat, dst_feat = pl.pallas_call(
        _kernel,
        grid=grid,
        in_specs=[
            pl.BlockSpec((BB, L), lambda i: (i, 0)),
            pl.BlockSpec((BB, L), lambda i: (i, 0)),
            wspec,
            wspec,
            pl.BlockSpec((D, D), lambda i: (0, 0)),
            wspec,
        ],
        out_specs=[
            pl.BlockSpec((BB, L, D), lambda i: (i, 0, 0)),
            pl.BlockSpec((BB, L, D), lambda i: (i, 0, 0)),
        ],
        out_shape=[out_shape, out_shape],
        compiler_params=pltpu.CompilerParams(
            dimension_semantics=("parallel",),
            vmem_limit_bytes=56 * 1024 * 1024,
        ),
    )(src_ids, dst_ids, W1, b1r, W2, b2r)
    return src_feat, dst_feat


# speedup vs baseline: 1.4356x; 1.1660x over previous
import jax
import jax.numpy as jnp
from jax.experimental import pallas as pl
from jax.experimental.pallas import tpu as pltpu

B, L, D = 256, 512, 128
BB = 8


def _kernel(s_ref, d_ref, w1_ref, b1_ref, w2_ref, b2_ref, out_s_ref, out_d_ref):
    w1_col = jnp.broadcast_to(jnp.transpose(w1_ref[...]), (D, L))
    b1_col = jnp.broadcast_to(jnp.transpose(b1_ref[...]), (D, L))
    w2b = w2_ref[...].astype(jnp.bfloat16)
    b2x2 = 2.0 * b2_ref[...]

    def counts(query, corpus):
        m = jnp.broadcast_to(query, (L, L)) == corpus
        return jnp.sum(jnp.where(m, 1.0, 0.0), axis=0, keepdims=True)

    def encode(c1, c2):
        u1 = jax.nn.relu(w1_col * jnp.broadcast_to(c1, (D, L)) + b1_col)
        u2 = jax.nn.relu(w1_col * jnp.broadcast_to(c2, (D, L)) + b1_col)
        ut = (u1 + u2).astype(jnp.bfloat16)
        out = jax.lax.dot_general(ut, w2b, (((0,), (0,)), ((), ())),
                                  preferred_element_type=jnp.float32)
        return out + b2x2

    for r in range(BB):
        srow = s_ref[r:r + 1, :]
        drow = d_ref[r:r + 1, :]
        s_corp = jnp.broadcast_to(jnp.transpose(srow), (L, L))
        d_corp = jnp.broadcast_to(jnp.transpose(drow), (L, L))
        s_q = jnp.where(srow == 0, -1, srow)
        d_q = jnp.where(drow == 0, -1, drow)

        c_ss = counts(s_q, s_corp)
        c_sd = counts(s_q, d_corp)
        c_ds = counts(d_q, s_corp)
        c_dd = counts(d_q, d_corp)

        out_s_ref[r] = encode(c_ss, c_sd)
        out_d_ref[r] = encode(c_ds, c_dd)


def kernel(src_ids, dst_ids, W1, b1, W2, b2):
    b1r = b1.reshape(1, D)
    b2r = b2.reshape(1, D)
    grid = (B // BB,)
    out_shape = jax.ShapeDtypeStruct((B, L, D), jnp.float32)
    wspec = pl.BlockSpec((1, D), lambda i: (0, 0))
    src_feat, dst_feat = pl.pallas_call(
        _kernel,
        grid=grid,
        in_specs=[
            pl.BlockSpec((BB, L), lambda i: (i, 0)),
            pl.BlockSpec((BB, L), lambda i: (i, 0)),
            wspec,
            wspec,
            pl.BlockSpec((D, D), lambda i: (0, 0)),
            wspec,
        ],
        out_specs=[
            pl.BlockSpec((BB, L, D), lambda i: (i, 0, 0)),
            pl.BlockSpec((BB, L, D), lambda i: (i, 0, 0)),
        ],
        out_shape=[out_shape, out_shape],
        compiler_params=pltpu.CompilerParams(
            dimension_semantics=("parallel",),
            vmem_limit_bytes=56 * 1024 * 1024,
        ),
    )(src_ids, dst_ids, W1, b1r, W2, b2r)
    return src_feat, dst_feat


# speedup vs baseline: 2.0717x; 1.4431x over previous
import jax
import jax.numpy as jnp
from jax.experimental import pallas as pl
from jax.experimental.pallas import tpu as pltpu

B, L, D = 256, 512, 128
BB = 8


def _kernel(s_ref, d_ref, w1_ref, b1_ref, w2_ref, b2_ref, out_s_ref, out_d_ref):
    w1_col = jnp.broadcast_to(jnp.transpose(w1_ref[...]), (D, L))
    b1_col = jnp.broadcast_to(jnp.transpose(b1_ref[...]), (D, L))
    w2b = w2_ref[...].astype(jnp.bfloat16)
    b2x2 = 2.0 * b2_ref[...]

    def counts(query, corpus):
        m = jnp.broadcast_to(query, (L, L)) == corpus
        x = jnp.where(m, jnp.int16(1), jnp.int16(0))
        s = L
        while s > 16:
            h = s // 2
            x = x[:h] + x[h:]
            s = h
        return jnp.sum(x.astype(jnp.float32), axis=0, keepdims=True)

    def encode(c1, c2):
        u1 = jax.nn.relu(w1_col * jnp.broadcast_to(c1, (D, L)) + b1_col)
        u2 = jax.nn.relu(w1_col * jnp.broadcast_to(c2, (D, L)) + b1_col)
        ut = (u1 + u2).astype(jnp.bfloat16)
        out = jax.lax.dot_general(ut, w2b, (((0,), (0,)), ((), ())),
                                  preferred_element_type=jnp.float32)
        return out + b2x2

    s16 = s_ref[...].astype(jnp.int16)
    d16 = d_ref[...].astype(jnp.int16)

    for r in range(BB):
        srow = s16[r:r + 1, :]
        drow = d16[r:r + 1, :]
        s_corp = jnp.broadcast_to(jnp.transpose(srow), (L, L))
        d_corp = jnp.broadcast_to(jnp.transpose(drow), (L, L))
        s_q = jnp.where(srow == jnp.int16(0), jnp.int16(-1), srow)
        d_q = jnp.where(drow == jnp.int16(0), jnp.int16(-1), drow)

        c_ss = counts(s_q, s_corp)
        c_sd = counts(s_q, d_corp)
        c_ds = counts(d_q, s_corp)
        c_dd = counts(d_q, d_corp)

        out_s_ref[r] = encode(c_ss, c_sd)
        out_d_ref[r] = encode(c_ds, c_dd)


def kernel(src_ids, dst_ids, W1, b1, W2, b2):
    b1r = b1.reshape(1, D)
    b2r = b2.reshape(1, D)
    grid = (B // BB,)
    out_shape = jax.ShapeDtypeStruct((B, L, D), jnp.float32)
    wspec = pl.BlockSpec((1, D), lambda i: (0, 0))
    src_feat, dst_feat = pl.pallas_call(
        _kernel,
        grid=grid,
        in_specs=[
            pl.BlockSpec((BB, L), lambda i: (i, 0)),
            pl.BlockSpec((BB, L), lambda i: (i, 0)),
            wspec,
            wspec,
            pl.BlockSpec((D, D), lambda i: (0, 0)),
            wspec,
        ],
        out_specs=[
            pl.BlockSpec((BB, L, D), lambda i: (i, 0, 0)),
            pl.BlockSpec((BB, L, D), lambda i: (i, 0, 0)),
        ],
        out_shape=[out_shape, out_shape],
        compiler_params=pltpu.CompilerParams(
            dimension_semantics=("parallel",),
            vmem_limit_bytes=56 * 1024 * 1024,
        ),
    )(src_ids, dst_ids, W1, b1r, W2, b2r)
    return src_feat, dst_feat


# speedup vs baseline: 2.2491x; 1.0856x over previous
import jax
import jax.numpy as jnp
from jax.experimental import pallas as pl
from jax.experimental.pallas import tpu as pltpu

B, L, D = 256, 512, 128
BB = 8


def _kernel(s_ref, d_ref, w1_ref, b1_ref, w2_ref, b2_ref, out_s_ref, out_d_ref):
    w1_col = jnp.broadcast_to(jnp.transpose(w1_ref[...]), (D, L)).astype(jnp.bfloat16)
    b1_col = jnp.broadcast_to(jnp.transpose(b1_ref[...]), (D, L)).astype(jnp.bfloat16)
    w2b = w2_ref[...].astype(jnp.bfloat16)
    b2x2 = 2.0 * b2_ref[...]

    CH = 128

    def counts(query, corpus_col):
        qb = jnp.broadcast_to(query, (CH, L))
        acc = None
        for c in range(0, L, CH):
            corp = jnp.broadcast_to(corpus_col[c:c + CH], (CH, L))
            x = jnp.where(qb == corp, jnp.int16(1), jnp.int16(0))
            s = CH
            while s > 16:
                h = s // 2
                x = x[:h] + x[h:]
                s = h
            acc = x if acc is None else acc + x
        return jnp.sum(acc.astype(jnp.float32), axis=0, keepdims=True)

    def encode(c1, c2):
        c1b = jnp.broadcast_to(c1.astype(jnp.bfloat16), (D, L))
        c2b = jnp.broadcast_to(c2.astype(jnp.bfloat16), (D, L))
        u1 = jax.nn.relu(w1_col * c1b + b1_col)
        u2 = jax.nn.relu(w1_col * c2b + b1_col)
        ut = u1 + u2
        out = jax.lax.dot_general(ut, w2b, (((0,), (0,)), ((), ())),
                                  preferred_element_type=jnp.float32)
        return out + b2x2

    s16 = s_ref[...].astype(jnp.int16)
    d16 = d_ref[...].astype(jnp.int16)

    for r in range(BB):
        srow = s16[r:r + 1, :]
        drow = d16[r:r + 1, :]
        s_corp = jnp.transpose(srow)
        d_corp = jnp.transpose(drow)
        s_q = jnp.where(srow == jnp.int16(0), jnp.int16(-1), srow)
        d_q = jnp.where(drow == jnp.int16(0), jnp.int16(-1), drow)

        c_ss = counts(s_q, s_corp)
        c_sd = counts(s_q, d_corp)
        c_ds = counts(d_q, s_corp)
        c_dd = counts(d_q, d_corp)

        out_s_ref[r] = encode(c_ss, c_sd)
        out_d_ref[r] = encode(c_ds, c_dd)


def kernel(src_ids, dst_ids, W1, b1, W2, b2):
    b1r = b1.reshape(1, D)
    b2r = b2.reshape(1, D)
    grid = (B // BB,)
    out_shape = jax.ShapeDtypeStruct((B, L, D), jnp.float32)
    wspec = pl.BlockSpec((1, D), lambda i: (0, 0))
    src_feat, dst_feat = pl.pallas_call(
        _kernel,
        grid=grid,
        in_specs=[
            pl.BlockSpec((BB, L), lambda i: (i, 0)),
            pl.BlockSpec((BB, L), lambda i: (i, 0)),
            wspec,
            wspec,
            pl.BlockSpec((D, D), lambda i: (0, 0)),
            wspec,
        ],
        out_specs=[
            pl.BlockSpec((BB, L, D), lambda i: (i, 0, 0)),
            pl.BlockSpec((BB, L, D), lambda i: (i, 0, 0)),
        ],
        out_shape=[out_shape, out_shape],
        compiler_params=pltpu.CompilerParams(
            dimension_semantics=("parallel",),
            vmem_limit_bytes=56 * 1024 * 1024,
        ),
    )(src_ids, dst_ids, W1, b1r, W2, b2r)
    return src_feat, dst_feat


# speedup vs baseline: 2.2880x; 1.0173x over previous
import jax
import jax.numpy as jnp
from jax.experimental import pallas as pl
from jax.experimental.pallas import tpu as pltpu

B, L, D = 256, 512, 128
BB = 16


def _kernel(s_ref, d_ref, w1_ref, b1_ref, w2_ref, b2_ref, out_s_ref, out_d_ref):
    w1_col = jnp.broadcast_to(jnp.transpose(w1_ref[...]), (D, L)).astype(jnp.bfloat16)
    b1_col = jnp.broadcast_to(jnp.transpose(b1_ref[...]), (D, L)).astype(jnp.bfloat16)
    w2b = w2_ref[...].astype(jnp.bfloat16)
    b2x2 = 2.0 * b2_ref[...]

    CH = 128

    def counts(query, corpus_col):
        qb = jnp.broadcast_to(query, (CH, L))
        acc = None
        for c in range(0, L, CH):
            corp = jnp.broadcast_to(corpus_col[c:c + CH], (CH, L))
            x = jnp.where(qb == corp, jnp.int16(1), jnp.int16(0))
            s = CH
            while s > 16:
                h = s // 2
                x = x[:h] + x[h:]
                s = h
            acc = x if acc is None else acc + x
        return jnp.sum(acc.astype(jnp.float32), axis=0, keepdims=True)

    def encode(c1, c2):
        c1b = jnp.broadcast_to(c1.astype(jnp.bfloat16), (D, L))
        c2b = jnp.broadcast_to(c2.astype(jnp.bfloat16), (D, L))
        u1 = jax.nn.relu(w1_col * c1b + b1_col)
        u2 = jax.nn.relu(w1_col * c2b + b1_col)
        ut = u1 + u2
        out = jax.lax.dot_general(ut, w2b, (((0,), (0,)), ((), ())),
                                  preferred_element_type=jnp.float32)
        return out + b2x2

    s16 = s_ref[...].astype(jnp.int16)
    d16 = d_ref[...].astype(jnp.int16)

    for r in range(BB):
        srow = s16[r:r + 1, :]
        drow = d16[r:r + 1, :]
        s_corp = jnp.transpose(srow)
        d_corp = jnp.transpose(drow)
        s_q = jnp.where(srow == jnp.int16(0), jnp.int16(-1), srow)
        d_q = jnp.where(drow == jnp.int16(0), jnp.int16(-1), drow)

        c_ss = counts(s_q, s_corp)
        c_sd = counts(s_q, d_corp)
        c_ds = counts(d_q, s_corp)
        c_dd = counts(d_q, d_corp)

        out_s_ref[r] = encode(c_ss, c_sd)
        out_d_ref[r] = encode(c_ds, c_dd)


def kernel(src_ids, dst_ids, W1, b1, W2, b2):
    b1r = b1.reshape(1, D)
    b2r = b2.reshape(1, D)
    grid = (B // BB,)
    out_shape = jax.ShapeDtypeStruct((B, L, D), jnp.float32)
    wspec = pl.BlockSpec((1, D), lambda i: (0, 0))
    src_feat, dst_feat = pl.pallas_call(
        _kernel,
        grid=grid,
        in_specs=[
            pl.BlockSpec((BB, L), lambda i: (i, 0)),
            pl.BlockSpec((BB, L), lambda i: (i, 0)),
            wspec,
            wspec,
            pl.BlockSpec((D, D), lambda i: (0, 0)),
            wspec,
        ],
        out_specs=[
            pl.BlockSpec((BB, L, D), lambda i: (i, 0, 0)),
            pl.BlockSpec((BB, L, D), lambda i: (i, 0, 0)),
        ],
        out_shape=[out_shape, out_shape],
        compiler_params=pltpu.CompilerParams(
            dimension_semantics=("parallel",),
            vmem_limit_bytes=56 * 1024 * 1024,
        ),
    )(src_ids, dst_ids, W1, b1r, W2, b2r)
    return src_feat, dst_feat
